# baseline (speedup 1.0000x reference)
import functools

import jax
import jax.numpy as jnp
from jax.experimental import pallas as pl
from jax.experimental.pallas import tpu as pltpu


def _mm_act_kernel(x_ref, w_ref, b_ref, o_ref, *, act):
    y = jax.lax.dot_general(x_ref[...], w_ref[...], (((1,), (0,)), ((), ())),
                            preferred_element_type=jnp.float32)
    y = y + b_ref[...]
    if act == "relu":
        y = jnp.maximum(y, 0.0)
    elif act == "sigmoid":
        y = jax.nn.sigmoid(y)
    o_ref[...] = y


def _matmul_act(x, w, b, act, bt=512):
    T, K = x.shape
    C = w.shape[1]
    bt = min(bt, T)
    nb = -(-T // bt)
    Tp = nb * bt
    if Tp != T:
        x = jnp.pad(x, ((0, Tp - T), (0, 0)))
    out = pl.pallas_call(
        functools.partial(_mm_act_kernel, act=act),
        grid=(nb,),
        in_specs=[
            pl.BlockSpec((bt, K), lambda i: (i, 0)),
            pl.BlockSpec((K, C), lambda i: (0, 0)),
            pl.BlockSpec((1, C), lambda i: (0, 0)),
        ],
        out_specs=pl.BlockSpec((bt, C), lambda i: (i, 0)),
        out_shape=jax.ShapeDtypeStruct((Tp, C), jnp.float32),
        compiler_params=pltpu.CompilerParams(dimension_semantics=("parallel",)),
    )(x, w, b.reshape(1, C))
    if Tp != T:
        out = out[:T]
    return out


def _conv_s2(x, w, b, act="relu", bt=512):
    N, H, W, Cin = x.shape
    Cout = w.shape[0]
    Ho, Wo = H // 2, W // 2
    xp = jnp.pad(x, ((0, 0), (1, 1), (1, 1), (0, 0)))
    slabs = [xp[:, di:di + 2 * Ho:2, dj:dj + 2 * Wo:2, :]
             for di in range(4) for dj in range(4)]
    xcol = jnp.concatenate(slabs, axis=-1).reshape(N * Ho * Wo, 16 * Cin)
    w2 = jnp.transpose(w, (2, 3, 1, 0)).reshape(16 * Cin, Cout)
    y = _matmul_act(xcol, w2, b, act, bt)
    return y.reshape(N, Ho, Wo, Cout)


_ROWTAPS = {0: ((0, 1), (-1, 3)), 1: ((0, 2), (1, 0))}


def _deconv_s2(z, w, b, act, bt=512):
    N, H, W, Cin = z.shape
    Cout = w.shape[1]
    zp = jnp.pad(z, ((0, 0), (1, 1), (1, 1), (0, 0)))
    outs = []
    for pa in (0, 1):
        for pb in (0, 1):
            slabs, wks = [], []
            for sa, da in _ROWTAPS[pa]:
                for sb, db in _ROWTAPS[pb]:
                    slabs.append(zp[:, 1 + sa:1 + sa + H, 1 + sb:1 + sb + W, :])
                    wks.append(w[:, :, da, db])
            xcol = jnp.concatenate(slabs, axis=-1).reshape(N * H * W, 4 * Cin)
            wp = jnp.concatenate(wks, axis=0)
            outs.append(_matmul_act(xcol, wp, b, act, bt).reshape(N, H, W, Cout))
    o = jnp.stack(outs, axis=0).reshape(2, 2, N, H, W, Cout)
    o = jnp.transpose(o, (2, 3, 0, 4, 1, 5)).reshape(N, 2 * H, 2 * W, Cout)
    return o


def _vq_kernel(x_ref, cb_ref, q_ref, idx_ref, part_ref, *, T, bt, K):
    x = x_ref[...]
    cb = cb_ref[...]
    d2 = (jnp.sum(x * x, axis=1, keepdims=True)
          - 2.0 * jax.lax.dot_general(x, cb, (((1,), (1,)), ((), ())),
                                      preferred_element_type=jnp.float32)
          + jnp.sum(cb * cb, axis=1)[None, :])
    dist = jnp.sqrt(jnp.maximum(d2, 0.0))
    mind = jnp.min(dist, axis=1, keepdims=True)
    lane = jax.lax.broadcasted_iota(jnp.int32, (bt, K), 1)
    idx = jnp.min(jnp.where(dist == mind, lane, K), axis=1)
    onehot = (lane == idx[:, None]).astype(jnp.float32)
    q = jax.lax.dot_general(onehot, cb, (((1,), (0,)), ((), ())),
                            preferred_element_type=jnp.float32)
    q_ref[...] = q
    idx_ref[...] = idx.reshape(1, 1, bt)
    tglob = (pl.program_id(0) * bt
             + jax.lax.broadcasted_iota(jnp.int32, (bt, 1), 0))
    part = jnp.sum(jnp.where(tglob < T, mind * mind, 0.0))
    part_ref[...] = jnp.broadcast_to(part.reshape(1, 1, 1), (1, 1, 128))


def _vq(flat, cb):
    T, C = flat.shape
    K = cb.shape[0]
    bt = 448 if T % 448 == 0 else min(512, T)
    nb = -(-T // bt)
    Tp = nb * bt
    if Tp != T:
        flat = jnp.pad(flat, ((0, Tp - T), (0, 0)))
    q, idx, part = pl.pallas_call(
        functools.partial(_vq_kernel, T=T, bt=bt, K=K),
        grid=(nb,),
        in_specs=[
            pl.BlockSpec((bt, C), lambda i: (i, 0)),
            pl.BlockSpec((K, C), lambda i: (0, 0)),
        ],
        out_specs=[
            pl.BlockSpec((bt, C), lambda i: (i, 0)),
            pl.BlockSpec((1, 1, bt), lambda i: (i, 0, 0)),
            pl.BlockSpec((1, 1, 128), lambda i: (i, 0, 0)),
        ],
        out_shape=[
            jax.ShapeDtypeStruct((Tp, C), jnp.float32),
            jax.ShapeDtypeStruct((nb, 1, bt), jnp.int32),
            jax.ShapeDtypeStruct((nb, 1, 128), jnp.float32),
        ],
        compiler_params=pltpu.CompilerParams(dimension_semantics=("parallel",)),
    )(flat, cb)
    return q[:T], idx.reshape(Tp)[:T], jnp.sum(part[:, 0, 0])


def kernel(x, ew1, eb1, ew2, eb2, ew3, eb3, codebook,
           dw1, db1, dw2, db2, dw3, db3):
    xh = jnp.transpose(x, (0, 2, 3, 1))
    h = _conv_s2(xh, ew1, eb1)
    h = _conv_s2(h, ew2, eb2)
    h = _conv_s2(h, ew3, eb3)
    N, Ho, Wo, C = h.shape
    flat = h.reshape(-1, C)
    q, _, sse = _vq(flat, codebook)
    loss = sse / float(flat.shape[0] * C)
    st = q.reshape(N, Ho, Wo, C)
    y = _deconv_s2(st, dw1, db1, "relu")
    y = _deconv_s2(y, dw2, db2, "relu")
    y = _deconv_s2(y, dw3, db3, "sigmoid")
    y = jnp.transpose(y, (0, 3, 1, 2))
    return (y, loss, loss)


# speedup vs baseline: 17.7400x; 17.7400x over previous
import functools

import jax
import jax.numpy as jnp
from jax.experimental import pallas as pl
from jax.experimental.pallas import tpu as pltpu

_ROWTAPS = {0: ((0, 1), (-1, 3)), 1: ((0, 2), (1, 0))}


def _conv1_kernel(x_ref, w_ref, b_ref, o_ref):
    slabs = []
    for dr in range(3):
        for dc in range(3):
            slabs.append(x_ref[0, pl.Slice(dr, 57, 2), pl.Slice(dc, 57, 2), :])
    xcol = jnp.concatenate(slabs, axis=-1)
    y = jax.lax.dot_general(xcol, w_ref[...], (((2,), (0,)), ((), ())),
                            preferred_element_type=jnp.float32)
    y = jnp.maximum(y + b_ref[...], 0.0)
    r = jax.lax.broadcasted_iota(jnp.int32, y.shape, 0)
    c = jax.lax.broadcasted_iota(jnp.int32, y.shape, 1)
    ch = jax.lax.broadcasted_iota(jnp.int32, y.shape, 2)
    y = jnp.where((r == 0) & (ch < 64), 0.0, y)
    y = jnp.where((r == 56) & (ch >= 64), 0.0, y)
    y = jnp.where((c == 0) & (ch % 64 < 32), 0.0, y)
    y = jnp.where((c == 56) & (ch % 64 >= 32), 0.0, y)
    o_ref[0, 1:58, 1:58, :] = y
    z_row = jnp.zeros((1, 60, 128), jnp.float32)
    z_col = jnp.zeros((60, 1, 128), jnp.float32)
    o_ref[0, 0:1, :, :] = z_row
    o_ref[0, 58:59, :, :] = z_row
    o_ref[0, 59:60, :, :] = z_row
    o_ref[0, :, 0:1, :] = z_col
    o_ref[0, :, 58:59, :] = z_col
    o_ref[0, :, 59:60, :] = z_col


def _conv2_kernel(x_ref, w_ref, b_ref, o_ref):
    for u in (0, 1):
        for uc in (0, 1):
            slabs = []
            for br in (0, 1):
                for bc in (0, 1):
                    slabs.append(x_ref[0, pl.Slice(u + br, 29, 2),
                                       pl.Slice(uc + bc, 29, 2), :])
            xcol = jnp.concatenate(slabs, axis=-1)
            y = jax.lax.dot_general(xcol, w_ref[...], (((2,), (0,)), ((), ())),
                                    preferred_element_type=jnp.float32)
            y = jnp.maximum(y + b_ref[...], 0.0)
            r = jax.lax.broadcasted_iota(jnp.int32, y.shape, 0)
            c = jax.lax.broadcasted_iota(jnp.int32, y.shape, 1)
            y = jnp.where(r == (0 if u == 0 else 28), 0.0, y)
            y = jnp.where(c == (0 if uc == 0 else 28), 0.0, y)
            o_ref[0, :, :, (u * 2 + uc) * 64:(u * 2 + uc) * 64 + 64] = y


def _conv3_kernel(x_ref, w_ref, b_ref, o_ref):
    slabs = []
    for br in (0, 1):
        for bc in (0, 1):
            slabs.append(x_ref[0, br:br + 28, bc:bc + 28, :])
    xcol = jnp.concatenate(slabs, axis=-1)
    y = jax.lax.dot_general(xcol, w_ref[...], (((2,), (0,)), ((), ())),
                            preferred_element_type=jnp.float32)
    o_ref[0] = jnp.maximum(y + b_ref[...], 0.0)


def _vq_kernel(x_ref, cb_ref, q_ref, part_ref):
    x = x_ref[0]
    cb = cb_ref[...]
    d2 = (jnp.sum(x * x, axis=2, keepdims=True)
          - 2.0 * jax.lax.dot_general(x, cb, (((2,), (1,)), ((), ())),
                                      preferred_element_type=jnp.float32)
          + jnp.sum(cb * cb, axis=1)[None, None, :])
    dist = jnp.sqrt(jnp.maximum(d2, 0.0))
    mind = jnp.min(dist, axis=2, keepdims=True)
    lane = jax.lax.broadcasted_iota(jnp.int32, dist.shape, 2)
    idx = jnp.min(jnp.where(dist == mind, lane, 512), axis=2, keepdims=True)
    onehot = (lane == idx).astype(jnp.float32)
    q = jax.lax.dot_general(onehot, cb, (((2,), (0,)), ((), ())),
                            preferred_element_type=jnp.float32)
    q_ref[0, 1:29, 1:29, :] = q
    z_row = jnp.zeros((1, 30, 64), jnp.float32)
    z_col = jnp.zeros((30, 1, 64), jnp.float32)
    q_ref[0, 0:1, :, :] = z_row
    q_ref[0, 29:30, :, :] = z_row
    q_ref[0, :, 0:1, :] = z_col
    q_ref[0, :, 29:30, :] = z_col
    part = jnp.sum(mind * mind)
    part_ref[...] = jnp.broadcast_to(part.reshape(1, 1, 1), (1, 1, 128))


def _deconv_kernel(x_ref, w_ref, b_ref, o_ref, *, Hi, Cout, act):
    Ho = 2 * Hi
    for pa in (0, 1):
        for pb in (0, 1):
            slabs = []
            for sa, _ in _ROWTAPS[pa]:
                for sb, _ in _ROWTAPS[pb]:
                    slabs.append(x_ref[0, 1 + sa:1 + sa + Hi,
                                       1 + sb:1 + sb + Hi, :])
            xcol = jnp.concatenate(slabs, axis=-1)
            y = jax.lax.dot_general(xcol, w_ref[pa, pb],
                                    (((2,), (0,)), ((), ())),
                                    preferred_element_type=jnp.float32)
            y = y + b_ref[...]
            y = jnp.maximum(y, 0.0) if act == "relu" else jax.nn.sigmoid(y)
            o_ref[0, pl.Slice(1 + pa, Hi, 2), pl.Slice(1 + pb, Hi, 2), :] = y
    z_row = jnp.zeros((1, Ho + 2, Cout), jnp.float32)
    z_col = jnp.zeros((Ho + 2, 1, Cout), jnp.float32)
    o_ref[0, 0:1, :, :] = z_row
    o_ref[0, Ho + 1:Ho + 2, :, :] = z_row
    o_ref[0, :, 0:1, :] = z_col
    o_ref[0, :, Ho + 1:Ho + 2, :] = z_col


def _deconv3_kernel(x_ref, w_ref, b_ref, o_ref):
    acc = None
    for dr in range(3):
        slabs = [x_ref[0, dr:dr + 112, dc:dc + 112, :] for dc in range(3)]
        xcol = jnp.concatenate(slabs, axis=-1)
        y = jax.lax.dot_general(xcol, w_ref[dr],
                                (((2,), (0,)), ((), ())),
                                preferred_element_type=jnp.float32)
        acc = y if acc is None else acc + y
    o_ref[0] = jax.nn.sigmoid(acc + b_ref[...])


def _per_image(kern, ins, in_shapes, out_shape, n=16):
    specs = [pl.BlockSpec((1,) + in_shapes[0][1:], lambda i: (i, 0, 0, 0))]
    for s in in_shapes[1:]:
        nd = len(s)
        specs.append(pl.BlockSpec(s, lambda i, _nd=nd: (0,) * _nd))
    return pl.pallas_call(
        kern,
        grid=(n,),
        in_specs=specs,
        out_specs=pl.BlockSpec((1,) + out_shape[1:], lambda i: (i, 0, 0, 0)),
        out_shape=jax.ShapeDtypeStruct(out_shape, jnp.float32),
        compiler_params=pltpu.CompilerParams(dimension_semantics=("parallel",)),
    )(*ins)


def _build_w1(ew1):
    w = jnp.zeros((3, 3, 2, 2, 3, 2, 2, 32), jnp.float32)
    for dr in range(3):
        for u in range(2):
            di = 2 * dr + 0 - 2 * u
            for dc in range(3):
                for uc in range(2):
                    dj = 2 * dc + 0 - 2 * uc
                    for q in range(2):
                        for qc in range(2):
                            a, b = di + q, dj + qc
                            if 0 <= a <= 3 and 0 <= b <= 3:
                                w = w.at[dr, dc, q, qc, :, u, uc, :].set(
                                    jnp.transpose(ew1[:, :, a, b], (1, 0)))
    return w.reshape(108, 128)


def _build_enc_w(ew):
    co, ci = ew.shape[0], ew.shape[1]
    w = ew.reshape(co, ci, 2, 2, 2, 2)
    w = jnp.transpose(w, (2, 4, 3, 5, 1, 0))
    return w.reshape(16 * ci, co)


def _build_dec_w(dw):
    ci, co = dw.shape[0], dw.shape[1]
    planes = []
    for pa in (0, 1):
        row = []
        for pb in (0, 1):
            taps = [dw[:, :, da, db]
                    for _, da in _ROWTAPS[pa] for _, db in _ROWTAPS[pb]]
            row.append(jnp.concatenate(taps, axis=0))
        planes.append(jnp.stack(row))
    return jnp.stack(planes)


def _build_w_d3(dw3):
    w = jnp.zeros((3, 3, 32, 2, 2, 3), jnp.float32)
    for pa in (0, 1):
        for sa, da in _ROWTAPS[pa]:
            for pb in (0, 1):
                for sb, db in _ROWTAPS[pb]:
                    w = w.at[sa + 1, sb + 1, :, pa, pb, :].set(dw3[:, :, da, db])
    return w.reshape(3, 96, 12)


def kernel(x, ew1, eb1, ew2, eb2, ew3, eb3, codebook,
           dw1, db1, dw2, db2, dw3, db3):
    n = x.shape[0]
    xh = jnp.pad(jnp.transpose(x, (0, 2, 3, 1)),
                 ((0, 0), (3, 3), (3, 3), (0, 0)))
    xs = jnp.transpose(xh.reshape(n, 115, 2, 115, 2, 3),
                       (0, 1, 3, 2, 4, 5)).reshape(n, 115, 115, 12)

    w1 = _build_w1(ew1)
    b1 = jnp.tile(eb1, 4).reshape(1, 1, 128)
    h1 = _per_image(_conv1_kernel, (xs, w1, b1),
                    ((n, 115, 115, 12), (108, 128), (1, 1, 128)),
                    (n, 60, 60, 128), n)

    w2 = _build_enc_w(ew2)
    b2 = eb2.reshape(1, 1, 64)
    h2 = _per_image(_conv2_kernel, (h1, w2, b2),
                    ((n, 60, 60, 128), (512, 64), (1, 1, 64)),
                    (n, 29, 29, 256), n)

    w3 = _build_enc_w(ew3)
    b3 = eb3.reshape(1, 1, 64)
    h3 = _per_image(_conv3_kernel, (h2, w3, b3),
                    ((n, 29, 29, 256), (1024, 64), (1, 1, 64)),
                    (n, 28, 28, 64), n)

    qp, part = pl.pallas_call(
        _vq_kernel,
        grid=(n,),
        in_specs=[pl.BlockSpec((1, 28, 28, 64), lambda i: (i, 0, 0, 0)),
                  pl.BlockSpec((512, 64), lambda i: (0, 0))],
        out_specs=[pl.BlockSpec((1, 30, 30, 64), lambda i: (i, 0, 0, 0)),
                   pl.BlockSpec((1, 1, 128), lambda i: (i, 0, 0))],
        out_shape=[jax.ShapeDtypeStruct((n, 30, 30, 64), jnp.float32),
                   jax.ShapeDtypeStruct((n, 1, 128), jnp.float32)],
        compiler_params=pltpu.CompilerParams(dimension_semantics=("parallel",)),
    )(h3, codebook)
    loss = jnp.sum(part[:, 0, 0]) / float(n * 28 * 28 * 64)

    wd1 = _build_dec_w(dw1)
    g1 = _per_image(functools.partial(_deconv_kernel, Hi=28, Cout=64,
                                      act="relu"),
                    (qp, wd1, db1.reshape(1, 1, 64)),
                    ((n, 30, 30, 64), (2, 2, 256, 64), (1, 1, 64)),
                    (n, 58, 58, 64), n)

    wd2 = _build_dec_w(dw2)
    g2 = _per_image(functools.partial(_deconv_kernel, Hi=56, Cout=32,
                                      act="relu"),
                    (g1, wd2, db2.reshape(1, 1, 32)),
                    ((n, 58, 58, 64), (2, 2, 256, 32), (1, 1, 32)),
                    (n, 114, 114, 32), n)

    wd3 = _build_w_d3(dw3)
    bd3 = jnp.tile(db3, 4).reshape(1, 1, 12)
    planes = _per_image(_deconv3_kernel, (g2, wd3, bd3),
                        ((n, 114, 114, 32), (3, 96, 12), (1, 1, 12)),
                        (n, 112, 112, 12), n)

    y = jnp.transpose(planes.reshape(n, 112, 112, 2, 2, 3),
                      (0, 5, 1, 3, 2, 4)).reshape(n, 3, 224, 224)
    return (y, loss, loss)


# speedup vs baseline: 20.7972x; 1.1723x over previous
import functools

import jax
import jax.numpy as jnp
from jax.experimental import pallas as pl
from jax.experimental.pallas import tpu as pltpu

_ROWTAPS = {0: ((0, 1), (-1, 3)), 1: ((0, 2), (1, 0))}


def _conv1_kernel(x_ref, w_ref, b_ref, o_ref):
    slabs = []
    for dr in range(3):
        for dc in range(3):
            slabs.append(x_ref[0, pl.Slice(dr, 57, 2), pl.Slice(dc, 57, 2), :])
    xcol = jnp.concatenate(slabs, axis=-1)
    y = jax.lax.dot_general(xcol, w_ref[...], (((2,), (0,)), ((), ())),
                            preferred_element_type=jnp.float32)
    y = jnp.maximum(y + b_ref[...], 0.0)
    r = jax.lax.broadcasted_iota(jnp.int32, y.shape, 0)
    c = jax.lax.broadcasted_iota(jnp.int32, y.shape, 1)
    ch = jax.lax.broadcasted_iota(jnp.int32, y.shape, 2)
    y = jnp.where((r == 0) & (ch < 64), 0.0, y)
    y = jnp.where((r == 56) & (ch >= 64), 0.0, y)
    y = jnp.where((c == 0) & (ch % 64 < 32), 0.0, y)
    y = jnp.where((c == 56) & (ch % 64 >= 32), 0.0, y)
    o_ref[0, 1:58, 1:58, :] = y
    z_row = jnp.zeros((1, 60, 128), jnp.float32)
    z_col = jnp.zeros((60, 1, 128), jnp.float32)
    o_ref[0, 0:1, :, :] = z_row
    o_ref[0, 58:59, :, :] = z_row
    o_ref[0, 59:60, :, :] = z_row
    o_ref[0, :, 0:1, :] = z_col
    o_ref[0, :, 58:59, :] = z_col
    o_ref[0, :, 59:60, :] = z_col


def _conv2_kernel(x_ref, w_ref, b_ref, o_ref):
    for u in (0, 1):
        for uc in (0, 1):
            slabs = []
            for br in (0, 1):
                for bc in (0, 1):
                    slabs.append(x_ref[0, pl.Slice(u + br, 29, 2),
                                       pl.Slice(uc + bc, 29, 2), :])
            xcol = jnp.concatenate(slabs, axis=-1)
            y = jax.lax.dot_general(xcol, w_ref[...], (((2,), (0,)), ((), ())),
                                    preferred_element_type=jnp.float32)
            y = jnp.maximum(y + b_ref[...], 0.0)
            r = jax.lax.broadcasted_iota(jnp.int32, y.shape, 0)
            c = jax.lax.broadcasted_iota(jnp.int32, y.shape, 1)
            y = jnp.where(r == (0 if u == 0 else 28), 0.0, y)
            y = jnp.where(c == (0 if uc == 0 else 28), 0.0, y)
            o_ref[0, :, :, (u * 2 + uc) * 64:(u * 2 + uc) * 64 + 64] = y


def _conv3_kernel(x_ref, w_ref, b_ref, o_ref):
    slabs = []
    for br in (0, 1):
        for bc in (0, 1):
            slabs.append(x_ref[0, br:br + 28, bc:bc + 28, :])
    xcol = jnp.concatenate(slabs, axis=-1)
    y = jax.lax.dot_general(xcol, w_ref[...], (((2,), (0,)), ((), ())),
                            preferred_element_type=jnp.float32)
    o_ref[0] = jnp.maximum(y + b_ref[...], 0.0)


def _vq_kernel(x_ref, cb_ref, q_ref, part_ref):
    x = x_ref[0]
    cb = cb_ref[...]
    d2 = (jnp.sum(x * x, axis=2, keepdims=True)
          - 2.0 * jax.lax.dot_general(x, cb, (((2,), (1,)), ((), ())),
                                      preferred_element_type=jnp.float32)
          + jnp.sum(cb * cb, axis=1)[None, None, :])
    dist = jnp.sqrt(jnp.maximum(d2, 0.0))
    mind = jnp.min(dist, axis=2, keepdims=True)
    lane = jax.lax.broadcasted_iota(jnp.int32, dist.shape, 2)
    idx = jnp.min(jnp.where(dist == mind, lane, 512), axis=2, keepdims=True)
    onehot = (lane == idx).astype(jnp.float32)
    q = jax.lax.dot_general(onehot, cb, (((2,), (0,)), ((), ())),
                            preferred_element_type=jnp.float32)
    q_ref[0, 1:29, 1:29, :] = q
    z_row = jnp.zeros((1, 30, 64), jnp.float32)
    z_col = jnp.zeros((30, 1, 64), jnp.float32)
    q_ref[0, 0:1, :, :] = z_row
    q_ref[0, 29:30, :, :] = z_row
    q_ref[0, :, 0:1, :] = z_col
    q_ref[0, :, 29:30, :] = z_col
    part = jnp.sum(mind * mind)
    part_ref[...] = jnp.broadcast_to(part.reshape(1, 1, 1), (1, 1, 128))


def _deconv_kernel(x_ref, w_ref, b_ref, o_ref, *, Hi, Cout, act):
    Ho = 2 * Hi
    for pa in (0, 1):
        for pb in (0, 1):
            slabs = []
            for sa, _ in _ROWTAPS[pa]:
                for sb, _ in _ROWTAPS[pb]:
                    slabs.append(x_ref[0, 1 + sa:1 + sa + Hi,
                                       1 + sb:1 + sb + Hi, :])
            xcol = jnp.concatenate(slabs, axis=-1)
            y = jax.lax.dot_general(xcol, w_ref[pa, pb],
                                    (((2,), (0,)), ((), ())),
                                    preferred_element_type=jnp.float32)
            y = y + b_ref[...]
            y = jnp.maximum(y, 0.0) if act == "relu" else jax.nn.sigmoid(y)
            o_ref[0, pl.Slice(1 + pa, Hi, 2), pl.Slice(1 + pb, Hi, 2), :] = y
    z_row = jnp.zeros((1, Ho + 2, Cout), jnp.float32)
    z_col = jnp.zeros((Ho + 2, 1, Cout), jnp.float32)
    o_ref[0, 0:1, :, :] = z_row
    o_ref[0, Ho + 1:Ho + 2, :, :] = z_row
    o_ref[0, :, 0:1, :] = z_col
    o_ref[0, :, Ho + 1:Ho + 2, :] = z_col


def _deconv2_kernel(x_ref, w_ref, b_ref, o_ref):
    for pa in (0, 1):
        for pb in (0, 1):
            slabs = []
            for sa, _ in _ROWTAPS[pa]:
                for sb, _ in _ROWTAPS[pb]:
                    slabs.append(x_ref[0, 1 + sa:1 + sa + 56,
                                       1 + sb:1 + sb + 56, :])
            xcol = jnp.concatenate(slabs, axis=-1)
            y = jax.lax.dot_general(xcol, w_ref[pa, pb],
                                    (((2,), (0,)), ((), ())),
                                    preferred_element_type=jnp.float32)
            y = jnp.maximum(y + b_ref[...], 0.0)
            ch0 = ((1 - pa) * 2 + (1 - pb)) * 32
            o_ref[0, pa:pa + 56, pb:pb + 56, ch0:ch0 + 32] = y
    o_ref[0, 0:1, :, 0:64] = jnp.zeros((1, 57, 64), jnp.float32)
    o_ref[0, 56:57, :, 64:128] = jnp.zeros((1, 57, 64), jnp.float32)
    z = jnp.zeros((57, 1, 32), jnp.float32)
    o_ref[0, :, 0:1, 0:32] = z
    o_ref[0, :, 0:1, 64:96] = z
    o_ref[0, :, 56:57, 32:64] = z
    o_ref[0, :, 56:57, 96:128] = z


def _deconv3_kernel(x_ref, w_ref, b_ref, o_ref):
    slabs = []
    for br in (0, 1):
        for bc in (0, 1):
            slabs.append(x_ref[0, br:br + 56, bc:bc + 56, :])
    xcol = jnp.concatenate(slabs, axis=-1)
    y = jax.lax.dot_general(xcol, w_ref[...], (((2,), (0,)), ((), ())),
                            preferred_element_type=jnp.float32)
    o_ref[0] = jax.nn.sigmoid(y + b_ref[...])


def _per_image(kern, ins, in_shapes, out_shape, n=16):
    specs = [pl.BlockSpec((1,) + in_shapes[0][1:], lambda i: (i, 0, 0, 0))]
    for s in in_shapes[1:]:
        nd = len(s)
        specs.append(pl.BlockSpec(s, lambda i, _nd=nd: (0,) * _nd))
    return pl.pallas_call(
        kern,
        grid=(n,),
        in_specs=specs,
        out_specs=pl.BlockSpec((1,) + out_shape[1:], lambda i: (i, 0, 0, 0)),
        out_shape=jax.ShapeDtypeStruct(out_shape, jnp.float32),
        compiler_params=pltpu.CompilerParams(dimension_semantics=("parallel",)),
    )(*ins)


def _build_w1(ew1):
    w = jnp.zeros((3, 3, 2, 2, 3, 2, 2, 32), jnp.float32)
    for dr in range(3):
        for u in range(2):
            di = 2 * dr + 0 - 2 * u
            for dc in range(3):
                for uc in range(2):
                    dj = 2 * dc + 0 - 2 * uc
                    for q in range(2):
                        for qc in range(2):
                            a, b = di + q, dj + qc
                            if 0 <= a <= 3 and 0 <= b <= 3:
                                w = w.at[dr, dc, q, qc, :, u, uc, :].set(
                                    jnp.transpose(ew1[:, :, a, b], (1, 0)))
    return w.reshape(108, 128)


def _build_enc_w(ew):
    co, ci = ew.shape[0], ew.shape[1]
    w = ew.reshape(co, ci, 2, 2, 2, 2)
    w = jnp.transpose(w, (2, 4, 3, 5, 1, 0))
    return w.reshape(16 * ci, co)


def _build_dec_w(dw):
    ci, co = dw.shape[0], dw.shape[1]
    planes = []
    for pa in (0, 1):
        row = []
        for pb in (0, 1):
            taps = [dw[:, :, da, db]
                    for _, da in _ROWTAPS[pa] for _, db in _ROWTAPS[pb]]
            row.append(jnp.concatenate(taps, axis=0))
        planes.append(jnp.stack(row))
    return jnp.stack(planes)


def _build_w_d3(dw3):
    w = jnp.zeros((2, 2, 2, 2, 32, 2, 2, 2, 2, 3), jnp.float32)
    taps = {p: dict(_ROWTAPS[p]) for p in (0, 1)}
    for br in (0, 1):
        for q in (0, 1):
            for v in (0, 1):
                for pa in (0, 1):
                    s = 2 * br + q - 1 - v
                    if s not in taps[pa]:
                        continue
                    da = taps[pa][s]
                    for bc in (0, 1):
                        for qc in (0, 1):
                            for vc in (0, 1):
                                for pb in (0, 1):
                                    t = 2 * bc + qc - 1 - vc
                                    if t not in taps[pb]:
                                        continue
                                    db = taps[pb][t]
                                    w = w.at[br, bc, q, qc, :,
                                             v, vc, pa, pb, :].set(
                                        dw3[:, :, da, db])
    return w.reshape(512, 48)


def kernel(x, ew1, eb1, ew2, eb2, ew3, eb3, codebook,
           dw1, db1, dw2, db2, dw3, db3):
    n = x.shape[0]
    xh = jnp.pad(jnp.transpose(x, (0, 2, 3, 1)),
                 ((0, 0), (3, 3), (3, 3), (0, 0)))
    xs = jnp.transpose(xh.reshape(n, 115, 2, 115, 2, 3),
                       (0, 1, 3, 2, 4, 5)).reshape(n, 115, 115, 12)

    w1 = _build_w1(ew1)
    b1 = jnp.tile(eb1, 4).reshape(1, 1, 128)
    h1 = _per_image(_conv1_kernel, (xs, w1, b1),
                    ((n, 115, 115, 12), (108, 128), (1, 1, 128)),
                    (n, 60, 60, 128), n)

    w2 = _build_enc_w(ew2)
    b2 = eb2.reshape(1, 1, 64)
    h2 = _per_image(_conv2_kernel, (h1, w2, b2),
                    ((n, 60, 60, 128), (512, 64), (1, 1, 64)),
                    (n, 29, 29, 256), n)

    w3 = _build_enc_w(ew3)
    b3 = eb3.reshape(1, 1, 64)
    h3 = _per_image(_conv3_kernel, (h2, w3, b3),
                    ((n, 29, 29, 256), (1024, 64), (1, 1, 64)),
                    (n, 28, 28, 64), n)

    qp, part = pl.pallas_call(
        _vq_kernel,
        grid=(n,),
        in_specs=[pl.BlockSpec((1, 28, 28, 64), lambda i: (i, 0, 0, 0)),
                  pl.BlockSpec((512, 64), lambda i: (0, 0))],
        out_specs=[pl.BlockSpec((1, 30, 30, 64), lambda i: (i, 0, 0, 0)),
                   pl.BlockSpec((1, 1, 128), lambda i: (i, 0, 0))],
        out_shape=[jax.ShapeDtypeStruct((n, 30, 30, 64), jnp.float32),
                   jax.ShapeDtypeStruct((n, 1, 128), jnp.float32)],
        compiler_params=pltpu.CompilerParams(dimension_semantics=("parallel",)),
    )(h3, codebook)
    loss = jnp.sum(part[:, 0, 0]) / float(n * 28 * 28 * 64)

    wd1 = _build_dec_w(dw1)
    g1 = _per_image(functools.partial(_deconv_kernel, Hi=28, Cout=64,
                                      act="relu"),
                    (qp, wd1, db1.reshape(1, 1, 64)),
                    ((n, 30, 30, 64), (2, 2, 256, 64), (1, 1, 64)),
                    (n, 58, 58, 64), n)

    wd2 = _build_dec_w(dw2)
    g2 = _per_image(_deconv2_kernel,
                    (g1, wd2, db2.reshape(1, 1, 32)),
                    ((n, 58, 58, 64), (2, 2, 256, 32), (1, 1, 32)),
                    (n, 57, 57, 128), n)

    wd3 = _build_w_d3(dw3)
    bd3 = jnp.tile(db3, 16).reshape(1, 1, 48)
    planes = _per_image(_deconv3_kernel, (g2, wd3, bd3),
                        ((n, 57, 57, 128), (512, 48), (1, 1, 48)),
                        (n, 56, 56, 48), n)

    y = jnp.transpose(planes.reshape(n, 56, 56, 2, 2, 2, 2, 3),
                      (0, 7, 1, 3, 5, 2, 4, 6)).reshape(n, 3, 224, 224)
    return (y, loss, loss)
